# speedup vs baseline: 3.4224x; 3.4224x over previous
import functools

import jax
import jax.numpy as jnp
from jax.experimental import pallas as pl
from jax.experimental.pallas import tpu as pltpu

S_BLK = 512


def _ln_body(tab_ref, w_ref, b_ref, out_ref):
    rows = tab_ref[...]
    mu = jnp.mean(rows, axis=1, keepdims=True)
    var = jnp.mean(rows * rows, axis=1, keepdims=True) - mu * mu
    normed = (rows - mu) * jax.lax.rsqrt(var + 1e-5)
    normed = normed * w_ref[...] + b_ref[...]
    t = normed.T
    for b in range(out_ref.shape[0]):
        out_ref[b] = t


@functools.partial(jax.jit, static_argnames=("seq_len", "batch"))
def _pos_embed(pos_table, ln_weight, ln_bias, seq_len, batch):
    d = pos_table.shape[1]
    grid = (seq_len // S_BLK,)
    return pl.pallas_call(
        _ln_body,
        grid=grid,
        in_specs=[
            pl.BlockSpec((S_BLK, d), lambda i: (i, 0)),
            pl.BlockSpec((1, d), lambda i: (0, 0)),
            pl.BlockSpec((1, d), lambda i: (0, 0)),
        ],
        out_specs=pl.BlockSpec((batch, d, S_BLK), lambda i: (0, 0, i)),
        out_shape=jax.ShapeDtypeStruct((batch, d, seq_len), pos_table.dtype),
    )(pos_table, ln_weight.reshape(1, d), ln_bias.reshape(1, d))


def kernel(x, pos_table, ln_weight, ln_bias):
    batch, _, seq_len = x.shape
    return _pos_embed(pos_table, ln_weight, ln_bias, seq_len, batch)
